# baseline (speedup 1.0000x reference)
import jax
import jax.numpy as jnp
from jax.experimental import pallas as pl
from jax.experimental.pallas import tpu as pltpu

NUM_EXPERTS = 64
HIDDEN = 1024
INTER = 512
TOKENS = 64
TOP_K = 8


EPB = 4


def _moe_body(x_ref, idx_ref, w_ref, gup_ref, down_ref, out_ref):
    step = pl.program_id(0)
    x = x_ref[...]
    acc = jnp.zeros((TOKENS, HIDDEN), jnp.float32)
    for i in range(EPB):
        e = step * EPB + i
        gup = gup_ref[i]
        gu = jax.lax.dot_general(
            x, gup, (((1,), (1,)), ((), ())),
            preferred_element_type=jnp.float32)
        gate = gu[:, :INTER]
        up = gu[:, INTER:]
        h = gate * jax.nn.sigmoid(gate) * up
        dwn = down_ref[i]
        out_e = jax.lax.dot_general(
            h, dwn, (((1,), (1,)), ((), ())),
            preferred_element_type=jnp.float32)
        sel = (idx_ref[...] == e).astype(jnp.float32)
        combine = jnp.sum(sel * w_ref[...], axis=1)
        acc = acc + out_e * combine[:, None]

    @pl.when(step == 0)
    def _init():
        out_ref[...] = acc

    @pl.when(step > 0)
    def _accum():
        out_ref[...] += acc


def kernel(hidden_states, top_k_index, top_k_weights, gate_up_proj, down_proj):
    return pl.pallas_call(
        _moe_body,
        grid=(NUM_EXPERTS // EPB,),
        in_specs=[
            pl.BlockSpec((TOKENS, HIDDEN), lambda e: (0, 0)),
            pl.BlockSpec((TOKENS, TOP_K), lambda e: (0, 0)),
            pl.BlockSpec((TOKENS, TOP_K), lambda e: (0, 0)),
            pl.BlockSpec((EPB, 2 * INTER, HIDDEN), lambda e: (e, 0, 0)),
            pl.BlockSpec((EPB, HIDDEN, INTER), lambda e: (e, 0, 0)),
        ],
        out_specs=pl.BlockSpec((TOKENS, HIDDEN), lambda e: (0, 0)),
        out_shape=jax.ShapeDtypeStruct((TOKENS, HIDDEN), jnp.float32),
        compiler_params=pltpu.CompilerParams(
            dimension_semantics=("arbitrary",),
        ),
    )(hidden_states, top_k_index, top_k_weights, gate_up_proj, down_proj)


# speedup vs baseline: 1.0493x; 1.0493x over previous
import jax
import jax.numpy as jnp
from jax.experimental import pallas as pl
from jax.experimental.pallas import tpu as pltpu

NUM_EXPERTS = 64
HIDDEN = 1024
INTER = 512
TOKENS = 64
TOP_K = 8

EPB = 2


def _moe_body(x_ref, idx_ref, w_ref, gate_w_ref, up_w_ref, dn0_ref, dn1_ref,
              out_ref):
    step = pl.program_id(0)
    x = x_ref[...]
    acc = jnp.zeros((TOKENS, HIDDEN), jnp.float32)
    for i in range(EPB):
        e = step * EPB + i
        gate = jax.lax.dot_general(
            x, gate_w_ref[i], (((1,), (1,)), ((), ())),
            preferred_element_type=jnp.float32)
        up = jax.lax.dot_general(
            x, up_w_ref[i], (((1,), (1,)), ((), ())),
            preferred_element_type=jnp.float32)
        h = gate * jax.nn.sigmoid(gate) * up
        out0 = jax.lax.dot_general(
            h, dn0_ref[i], (((1,), (1,)), ((), ())),
            preferred_element_type=jnp.float32)
        out1 = jax.lax.dot_general(
            h, dn1_ref[i], (((1,), (1,)), ((), ())),
            preferred_element_type=jnp.float32)
        out_e = jnp.concatenate([out0, out1], axis=1)
        sel = (idx_ref[...] == e).astype(jnp.float32)
        combine = jnp.sum(sel * w_ref[...], axis=1)
        acc = acc + out_e * combine[:, None]

    @pl.when(step == 0)
    def _init():
        out_ref[...] = acc

    @pl.when(step > 0)
    def _accum():
        out_ref[...] += acc


def kernel(hidden_states, top_k_index, top_k_weights, gate_up_proj, down_proj):
    return pl.pallas_call(
        _moe_body,
        grid=(NUM_EXPERTS // EPB,),
        in_specs=[
            pl.BlockSpec((TOKENS, HIDDEN), lambda e: (0, 0)),
            pl.BlockSpec((TOKENS, TOP_K), lambda e: (0, 0)),
            pl.BlockSpec((TOKENS, TOP_K), lambda e: (0, 0)),
            pl.BlockSpec((EPB, INTER, HIDDEN), lambda e: (e, 0, 0)),
            pl.BlockSpec((EPB, INTER, HIDDEN), lambda e: (e, 1, 0)),
            pl.BlockSpec((EPB, HIDDEN // 2, INTER), lambda e: (e, 0, 0)),
            pl.BlockSpec((EPB, HIDDEN // 2, INTER), lambda e: (e, 1, 0)),
        ],
        out_specs=pl.BlockSpec((TOKENS, HIDDEN), lambda e: (0, 0)),
        out_shape=jax.ShapeDtypeStruct((TOKENS, HIDDEN), jnp.float32),
        compiler_params=pltpu.CompilerParams(
            dimension_semantics=("arbitrary",),
        ),
    )(hidden_states, top_k_index, top_k_weights,
      gate_up_proj, gate_up_proj, down_proj, down_proj)
